# baseline (speedup 1.0000x reference)
import math
import numpy as np
import jax
import jax.numpy as jnp
from jax import lax
from jax.experimental import pallas as pl
from jax.experimental.pallas import tpu as pltpu
from jax.experimental.pallas import tpu_sc as plsc

_B = 16
_TOTAL_STEPS = 8
_CURRENT_STEP = 4
_LP = 512
_LA = 1024
_VOCAB = 1024
_D = 1024
_LANES = 16
_CHUNK = 32


def _embed_kernel(prompts, tokens, table, out,
                  ipx, itx, ib0, ib1, acc_v, db0, db1, sem0, sem1):
    nc = 2
    wid = lax.axis_index("s") * nc + lax.axis_index("c")
    b = wid // 2
    half = wid % 2
    scale = math.sqrt(float(_D))
    dbufs = (db0, db1)
    ibufs = (ib0, ib1)
    sems = (sem0, sem1)
    himask = jnp.full((_LANES,), np.int32(-65536), dtype=jnp.int32)
    shift16 = jnp.full((_LANES,), 16, dtype=jnp.int32)

    def stage(dst, idx_v, i, c0):
        for j in range(0, _CHUNK, _LANES):
            dst[pl.ds(j, _LANES)] = idx_v[i, pl.ds(c0 + j, _LANES)]

    def body_row(buf, mode, r, _):
        for k in range(0, _D // 2, _LANES):
            w = buf[r, pl.ds(k, _LANES)]
            lo = lax.bitcast_convert_type(lax.shift_left(w, shift16), jnp.float32)
            hi = lax.bitcast_convert_type(lax.bitwise_and(w, himask), jnp.float32)
            slo = pl.ds(2 * k, _LANES)
            shi = pl.ds(2 * k + _LANES, _LANES)
            if mode == 0:
                acc_v[r, slo] = lo
                acc_v[r, shi] = hi
            elif mode == 1:
                plsc.addupdate(acc_v.at[r, slo], lo)
                plsc.addupdate(acc_v.at[r, shi], hi)
            else:
                acc_v[r, slo] = (acc_v[r, slo] + lo) * scale
                acc_v[r, shi] = (acc_v[r, shi] + hi) * scale
        return 0

    def do_section(idx_hbm, idx_v, num_steps, t0, nrows, out_t0):

        pltpu.sync_copy(idx_hbm.at[b, :, pl.ds(t0, nrows)], idx_v)
        for i in range(1, num_steps):
            off = jnp.full((_LANES,), i * _VOCAB, dtype=jnp.int32)
            for j in range(0, nrows, _LANES):
                sl = pl.ds(j, _LANES)
                idx_v[i, sl] = idx_v[i, sl] + off

        def chunk_body(ci, _):
            c0 = pl.multiple_of(ci * _CHUNK, _CHUNK)
            stage(ibufs[0], idx_v, 0, c0)
            caps = {0: pltpu.async_copy(table.at[ibufs[0]], dbufs[0], sems[0])}
            stage(ibufs[1], idx_v, 1, c0)
            caps[1] = pltpu.async_copy(table.at[ibufs[1]], dbufs[1], sems[1])
            for i in range(num_steps):
                s = i % 2
                caps[i].wait()
                mode = 0 if i == 0 else (2 if i == num_steps - 1 else 1)
                lax.fori_loop(
                    0, _CHUNK,
                    lambda r, c, _buf=dbufs[s], _m=mode: body_row(_buf, _m, r, c),
                    0)
                if i + 2 < num_steps:
                    stage(ibufs[s], idx_v, i + 2, c0)
                    caps[i + 2] = pltpu.async_copy(
                        table.at[ibufs[s]], dbufs[s], sems[s])
            pltpu.sync_copy(acc_v, out.at[b, pl.ds(out_t0 + t0 + c0, _CHUNK)])
            return 0

        lax.fori_loop(0, nrows // _CHUNK, chunk_body, 0)

    do_section(prompts, ipx, _TOTAL_STEPS, half * (_LP // 2), _LP // 2, 0)
    do_section(tokens, itx, _CURRENT_STEP, half * (_LA // 2), _LA // 2, _LP)


def _pair_permutation():
    src = np.empty(_D, np.int32)
    for j in range(_D // 32):
        for t in range(16):
            src[32 * j + 2 * t] = 32 * j + t
            src[32 * j + 2 * t + 1] = 32 * j + 16 + t
    return src


def kernel(acoustic_prompts, acoustic_tokens, a_embeds):
    b, total_steps, lp = acoustic_prompts.shape
    current_step = acoustic_tokens.shape[1]
    la = acoustic_tokens.shape[2]
    d = a_embeds.shape[-1]

    prompts = acoustic_prompts.astype(jnp.int32)
    tokens = acoustic_tokens.astype(jnp.int32)
    table = a_embeds.reshape(total_steps * _VOCAB, d).astype(jnp.bfloat16)
    table = table[:, _pair_permutation()].reshape(total_steps * _VOCAB, d // 2, 2)
    table = lax.bitcast_convert_type(table, jnp.int32)

    mesh = plsc.VectorSubcoreMesh(
        core_axis_name="c", subcore_axis_name="s", num_cores=2, num_subcores=16
    )
    embeds = pl.kernel(
        _embed_kernel,
        out_type=jax.ShapeDtypeStruct((b, lp + la, d), jnp.float32),
        mesh=mesh,
        scratch_types=[
            pltpu.VMEM((_TOTAL_STEPS, _LP // 2), jnp.int32),
            pltpu.VMEM((_CURRENT_STEP, _LA // 2), jnp.int32),
            pltpu.VMEM((_CHUNK,), jnp.int32),
            pltpu.VMEM((_CHUNK,), jnp.int32),
            pltpu.VMEM((_CHUNK, _D), jnp.float32),
            pltpu.VMEM((_CHUNK, _D // 2), jnp.int32),
            pltpu.VMEM((_CHUNK, _D // 2), jnp.int32),
            pltpu.SemaphoreType.DMA,
            pltpu.SemaphoreType.DMA,
        ],
    )(prompts, tokens, table)

    seq_len = lp + la
    seq_lens = jnp.full((b,), seq_len, dtype=jnp.int32)
    padding_mask = jnp.arange(seq_len)[None, :] >= seq_lens[:, None]
    return embeds, padding_mask, current_step - 1


# speedup vs baseline: 1.0904x; 1.0904x over previous
import math
import numpy as np
import jax
import jax.numpy as jnp
from jax import lax
from jax.experimental import pallas as pl
from jax.experimental.pallas import tpu as pltpu
from jax.experimental.pallas import tpu_sc as plsc

_B = 16
_TOTAL_STEPS = 8
_CURRENT_STEP = 4
_LP = 512
_LA = 1024
_VOCAB = 1024
_D = 1024
_LANES = 16
_CHUNK = 32


def _embed_kernel(prompts, tokens, table, out,
                  ipx, itx, ib0, ib1, acc_v, db0, db1, sem0, sem1):
    nc = 2
    wid = lax.axis_index("s") * nc + lax.axis_index("c")
    b = wid // 2
    half = wid % 2
    scale = math.sqrt(float(_D))
    dbufs = (db0, db1)
    ibufs = (ib0, ib1)
    sems = (sem0, sem1)
    himask = jnp.full((_LANES,), np.int32(-65536), dtype=jnp.int32)
    shift16 = jnp.full((_LANES,), 16, dtype=jnp.int32)
    iota2 = lax.iota(jnp.int32, _LANES) * 2

    def stage(dst, idx_v, i, c0):
        for j in range(0, _CHUNK, _LANES):
            dst[pl.ds(j, _LANES)] = idx_v[i, pl.ds(c0 + j, _LANES)]

    def body_row(buf, mode, r, _):
        for k in range(0, _D // 2, _LANES):
            w = buf[r, pl.ds(k, _LANES)]
            lo = lax.bitcast_convert_type(lax.shift_left(w, shift16), jnp.float32)
            hi = lax.bitcast_convert_type(lax.bitwise_and(w, himask), jnp.float32)
            slo = pl.ds(2 * k, _LANES)
            shi = pl.ds(2 * k + _LANES, _LANES)
            if mode == 0:
                acc_v[r, slo] = lo
                acc_v[r, shi] = hi
            elif mode == 1:
                plsc.addupdate(acc_v.at[r, slo], lo)
                plsc.addupdate(acc_v.at[r, shi], hi)
            else:
                acc_v[r, slo] = (acc_v[r, slo] + lo) * scale
                acc_v[r, shi] = (acc_v[r, shi] + hi) * scale
        return 0

    def do_section(idx_hbm, idx_v, num_steps, t0, nrows, out_t0):

        pltpu.sync_copy(idx_hbm.at[b, :, pl.ds(t0, nrows)], idx_v)
        for i in range(1, num_steps):
            off = jnp.full((_LANES,), i * _VOCAB, dtype=jnp.int32)
            for j in range(0, nrows, _LANES):
                sl = pl.ds(j, _LANES)
                idx_v[i, sl] = idx_v[i, sl] + off

        def chunk_body(ci, _):
            c0 = pl.multiple_of(ci * _CHUNK, _CHUNK)
            stage(ibufs[0], idx_v, 0, c0)
            caps = {0: pltpu.async_copy(table.at[ibufs[0]], dbufs[0], sems[0])}
            stage(ibufs[1], idx_v, 1, c0)
            caps[1] = pltpu.async_copy(table.at[ibufs[1]], dbufs[1], sems[1])
            for i in range(num_steps):
                s = i % 2
                caps[i].wait()
                mode = 0 if i == 0 else (2 if i == num_steps - 1 else 1)
                lax.fori_loop(
                    0, _CHUNK,
                    lambda r, c, _buf=dbufs[s], _m=mode: body_row(_buf, _m, r, c),
                    0)
                if i + 2 < num_steps:
                    stage(ibufs[s], idx_v, i + 2, c0)
                    caps[i + 2] = pltpu.async_copy(
                        table.at[ibufs[s]], dbufs[s], sems[s])
            pltpu.sync_copy(acc_v, out.at[b, pl.ds(out_t0 + t0 + c0, _CHUNK)])
            return 0

        lax.fori_loop(0, nrows // _CHUNK, chunk_body, 0)

    do_section(prompts, ipx, _TOTAL_STEPS, half * (_LP // 2), _LP // 2, 0)
    do_section(tokens, itx, _CURRENT_STEP, half * (_LA // 2), _LA // 2, _LP)


def kernel(acoustic_prompts, acoustic_tokens, a_embeds):
    b, total_steps, lp = acoustic_prompts.shape
    current_step = acoustic_tokens.shape[1]
    la = acoustic_tokens.shape[2]
    d = a_embeds.shape[-1]

    prompts = acoustic_prompts.astype(jnp.int32)
    tokens = acoustic_tokens.astype(jnp.int32)
    table = a_embeds.reshape(total_steps * _VOCAB, d).astype(jnp.bfloat16)
    table = table.reshape(total_steps * _VOCAB, d // 32, 2, _LANES)
    table = table.swapaxes(2, 3).reshape(total_steps * _VOCAB, d // 2, 2)
    table = lax.bitcast_convert_type(table, jnp.int32)

    mesh = plsc.VectorSubcoreMesh(
        core_axis_name="c", subcore_axis_name="s", num_cores=2, num_subcores=16
    )
    embeds = pl.kernel(
        _embed_kernel,
        out_type=jax.ShapeDtypeStruct((b, lp + la, d), jnp.float32),
        mesh=mesh,
        scratch_types=[
            pltpu.VMEM((_TOTAL_STEPS, _LP // 2), jnp.int32),
            pltpu.VMEM((_CURRENT_STEP, _LA // 2), jnp.int32),
            pltpu.VMEM((_CHUNK,), jnp.int32),
            pltpu.VMEM((_CHUNK,), jnp.int32),
            pltpu.VMEM((_CHUNK, _D), jnp.float32),
            pltpu.VMEM((_CHUNK, _D // 2), jnp.int32),
            pltpu.VMEM((_CHUNK, _D // 2), jnp.int32),
            pltpu.SemaphoreType.DMA,
            pltpu.SemaphoreType.DMA,
        ],
    )(prompts, tokens, table)

    seq_len = lp + la
    seq_lens = jnp.full((b,), seq_len, dtype=jnp.int32)
    padding_mask = jnp.arange(seq_len)[None, :] >= seq_lens[:, None]
    return embeds, padding_mask, current_step - 1


# speedup vs baseline: 1.0938x; 1.0031x over previous
import math
import numpy as np
import jax
import jax.numpy as jnp
from jax import lax
from jax.experimental import pallas as pl
from jax.experimental.pallas import tpu as pltpu
from jax.experimental.pallas import tpu_sc as plsc

_B = 16
_TOTAL_STEPS = 8
_CURRENT_STEP = 4
_LP = 512
_LA = 1024
_VOCAB = 1024
_D = 1024
_LANES = 16
_CHUNK = 32


def _embed_kernel(prompts, tokens, table, out,
                  ipx, itx, ib0, ib1, acc_v, db0, db1, sem0, sem1):
    nc = 2
    wid = lax.axis_index("s") * nc + lax.axis_index("c")
    b = wid // 2
    half = wid % 2
    scale = math.sqrt(float(_D))
    dbufs = (db0, db1)
    ibufs = (ib0, ib1)
    sems = (sem0, sem1)
    himask = jnp.full((_LANES,), np.int32(-65536), dtype=jnp.int32)
    shift16 = jnp.full((_LANES,), 16, dtype=jnp.int32)
    iota2 = lax.iota(jnp.int32, _LANES) * 2

    def stage(dst, idx_v, i, c0):
        for j in range(0, _CHUNK, _LANES):
            dst[pl.ds(j, _LANES)] = idx_v[i, pl.ds(c0 + j, _LANES)]

    def body_row(buf, mode, r, _):
        for k in range(0, _D // 2, _LANES):
            w = buf[r, pl.ds(k, _LANES)]
            lo = lax.bitcast_convert_type(lax.shift_left(w, shift16), jnp.float32)
            hi = lax.bitcast_convert_type(lax.bitwise_and(w, himask), jnp.float32)
            slo = pl.ds(2 * k, _LANES)
            shi = pl.ds(2 * k + _LANES, _LANES)
            if mode == 0:
                acc_v[r, slo] = lo
                acc_v[r, shi] = hi
            elif mode == 1:
                plsc.addupdate(acc_v.at[r, slo], lo)
                plsc.addupdate(acc_v.at[r, shi], hi)
            else:
                acc_v[r, slo] = (acc_v[r, slo] + lo) * scale
                acc_v[r, shi] = (acc_v[r, shi] + hi) * scale
        return 0

    def do_section(idx_hbm, idx_v, num_steps, t0, nrows, out_t0):

        pltpu.sync_copy(idx_hbm.at[b, :, pl.ds(t0, nrows)], idx_v)
        for i in range(1, num_steps):
            off = jnp.full((_LANES,), i * _VOCAB, dtype=jnp.int32)
            for j in range(0, nrows, _LANES):
                sl = pl.ds(j, _LANES)
                idx_v[i, sl] = idx_v[i, sl] + off

        def chunk_body(ci, _):
            c0 = pl.multiple_of(ci * _CHUNK, _CHUNK)
            stage(ibufs[0], idx_v, 0, c0)
            caps = {0: pltpu.async_copy(table.at[ibufs[0]], dbufs[0], sems[0])}
            stage(ibufs[1], idx_v, 1, c0)
            caps[1] = pltpu.async_copy(table.at[ibufs[1]], dbufs[1], sems[1])
            for i in range(num_steps):
                s = i % 2
                caps[i].wait()
                mode = 0 if i == 0 else (2 if i == num_steps - 1 else 1)
                lax.fori_loop(
                    0, _CHUNK,
                    lambda r, c, _buf=dbufs[s], _m=mode: body_row(_buf, _m, r, c),
                    0)
                if i + 2 < num_steps:
                    stage(ibufs[s], idx_v, i + 2, c0)
                    caps[i + 2] = pltpu.async_copy(
                        table.at[ibufs[s]], dbufs[s], sems[s])
            pltpu.sync_copy(acc_v, out.at[b, pl.ds(out_t0 + t0 + c0, _CHUNK)])
            return 0

        lax.fori_loop(0, nrows // _CHUNK, chunk_body, 0)

    do_section(prompts, ipx, _TOTAL_STEPS, half * (_LP // 2), _LP // 2, 0)
    do_section(tokens, itx, _CURRENT_STEP, half * (_LA // 2), _LA // 2, _LP)


def kernel(acoustic_prompts, acoustic_tokens, a_embeds):
    b, total_steps, lp = acoustic_prompts.shape
    current_step = acoustic_tokens.shape[1]
    la = acoustic_tokens.shape[2]
    d = a_embeds.shape[-1]

    prompts = acoustic_prompts.astype(jnp.int32)
    tokens = acoustic_tokens.astype(jnp.int32)
    v = total_steps * _VOCAB
    tbits = lax.bitcast_convert_type(a_embeds.reshape(v, d), jnp.int32)
    rv = lax.shift_right_logical(
        tbits + 0x7FFF + (lax.shift_right_logical(tbits, 16) & 1), 16)
    rv = rv.reshape(v, d // 32, 2, _LANES)
    table = rv[:, :, 0, :] | lax.shift_left(rv[:, :, 1, :], 16)
    table = table.reshape(v, d // 2)

    mesh = plsc.VectorSubcoreMesh(
        core_axis_name="c", subcore_axis_name="s", num_cores=2, num_subcores=16
    )
    embeds = pl.kernel(
        _embed_kernel,
        out_type=jax.ShapeDtypeStruct((b, lp + la, d), jnp.float32),
        mesh=mesh,
        scratch_types=[
            pltpu.VMEM((_TOTAL_STEPS, _LP // 2), jnp.int32),
            pltpu.VMEM((_CURRENT_STEP, _LA // 2), jnp.int32),
            pltpu.VMEM((_CHUNK,), jnp.int32),
            pltpu.VMEM((_CHUNK,), jnp.int32),
            pltpu.VMEM((_CHUNK, _D), jnp.float32),
            pltpu.VMEM((_CHUNK, _D // 2), jnp.int32),
            pltpu.VMEM((_CHUNK, _D // 2), jnp.int32),
            pltpu.SemaphoreType.DMA,
            pltpu.SemaphoreType.DMA,
        ],
    )(prompts, tokens, table)

    seq_len = lp + la
    seq_lens = jnp.full((b,), seq_len, dtype=jnp.int32)
    padding_mask = jnp.arange(seq_len)[None, :] >= seq_lens[:, None]
    return embeds, padding_mask, current_step - 1


# speedup vs baseline: 2.0563x; 1.8800x over previous
import math
import numpy as np
import jax
import jax.numpy as jnp
from jax import lax
from jax.experimental import pallas as pl
from jax.experimental.pallas import tpu as pltpu
from jax.experimental.pallas import tpu_sc as plsc

_B = 16
_TOTAL_STEPS = 8
_CURRENT_STEP = 4
_LP = 512
_LA = 1024
_VOCAB = 1024
_D = 1024
_LANES = 16
_CHUNK = 32


def _embed_kernel(prompts, tokens, table, out,
                  ipx, itx, ib0, ib1, acc_v, db0, db1, sem0, sem1):
    nc = 2
    wid = lax.axis_index("s") * nc + lax.axis_index("c")
    b = wid // 2
    half = wid % 2
    scale = math.sqrt(float(_D))
    dbufs = (db0, db1)
    ibufs = (ib0, ib1)
    sems = (sem0, sem1)
    himask = jnp.full((_LANES,), np.int32(-65536), dtype=jnp.int32)
    shift16 = jnp.full((_LANES,), 16, dtype=jnp.int32)
    iota2 = lax.iota(jnp.int32, _LANES) * 2

    def stage(dst, idx_v, i, c0):
        for j in range(0, _CHUNK, _LANES):
            dst[pl.ds(j, _LANES)] = idx_v[i, pl.ds(c0 + j, _LANES)]

    def body_row(buf, mode, r, _):
        for k in range(0, _D // 2, _LANES):
            w = buf[r, pl.ds(k, _LANES)]
            lo = lax.bitcast_convert_type(lax.shift_left(w, shift16), jnp.float32)
            hi = lax.bitcast_convert_type(lax.bitwise_and(w, himask), jnp.float32)
            slo = pl.ds(2 * k, _LANES)
            shi = pl.ds(2 * k + _LANES, _LANES)
            if mode == 0:
                acc_v[r, slo] = lo
                acc_v[r, shi] = hi
            elif mode == 1:
                plsc.addupdate(acc_v.at[r, slo], lo)
                plsc.addupdate(acc_v.at[r, shi], hi)
            else:
                acc_v[r, slo] = (acc_v[r, slo] + lo) * scale
                acc_v[r, shi] = (acc_v[r, shi] + hi) * scale
        return 0

    def do_section(idx_hbm, idx_v, num_steps, t0, nrows, out_t0):

        pltpu.sync_copy(idx_hbm.at[b, :, pl.ds(t0, nrows)], idx_v)
        for i in range(1, num_steps):
            off = jnp.full((_LANES,), i * _VOCAB, dtype=jnp.int32)
            for j in range(0, nrows, _LANES):
                sl = pl.ds(j, _LANES)
                idx_v[i, sl] = idx_v[i, sl] + off

        def chunk_body(ci, _):
            c0 = pl.multiple_of(ci * _CHUNK, _CHUNK)
            stage(ibufs[0], idx_v, 0, c0)
            caps = {0: pltpu.async_copy(table.at[ibufs[0]], dbufs[0], sems[0])}
            stage(ibufs[1], idx_v, 1, c0)
            caps[1] = pltpu.async_copy(table.at[ibufs[1]], dbufs[1], sems[1])
            for i in range(num_steps):
                s = i % 2
                caps[i].wait()
                mode = 0 if i == 0 else (2 if i == num_steps - 1 else 1)
                def _pass(r, _buf=dbufs[s], _m=mode):
                    body_row(_buf, _m, r, 0)
                plsc.parallel_loop(0, _CHUNK, step=1)(_pass)
                if i + 2 < num_steps:
                    stage(ibufs[s], idx_v, i + 2, c0)
                    caps[i + 2] = pltpu.async_copy(
                        table.at[ibufs[s]], dbufs[s], sems[s])
            pltpu.sync_copy(acc_v, out.at[b, pl.ds(out_t0 + t0 + c0, _CHUNK)])
            return 0

        lax.fori_loop(0, nrows // _CHUNK, chunk_body, 0)

    do_section(prompts, ipx, _TOTAL_STEPS, half * (_LP // 2), _LP // 2, 0)
    do_section(tokens, itx, _CURRENT_STEP, half * (_LA // 2), _LA // 2, _LP)


def kernel(acoustic_prompts, acoustic_tokens, a_embeds):
    b, total_steps, lp = acoustic_prompts.shape
    current_step = acoustic_tokens.shape[1]
    la = acoustic_tokens.shape[2]
    d = a_embeds.shape[-1]

    prompts = acoustic_prompts.astype(jnp.int32)
    tokens = acoustic_tokens.astype(jnp.int32)
    v = total_steps * _VOCAB
    tbits = lax.bitcast_convert_type(a_embeds.reshape(v, d), jnp.int32)
    rv = lax.shift_right_logical(
        tbits + 0x7FFF + (lax.shift_right_logical(tbits, 16) & 1), 16)
    rv = rv.reshape(v, d // 32, 2, _LANES)
    table = rv[:, :, 0, :] | lax.shift_left(rv[:, :, 1, :], 16)
    table = table.reshape(v, d // 2)

    mesh = plsc.VectorSubcoreMesh(
        core_axis_name="c", subcore_axis_name="s", num_cores=2, num_subcores=16
    )
    embeds = pl.kernel(
        _embed_kernel,
        out_type=jax.ShapeDtypeStruct((b, lp + la, d), jnp.float32),
        mesh=mesh,
        scratch_types=[
            pltpu.VMEM((_TOTAL_STEPS, _LP // 2), jnp.int32),
            pltpu.VMEM((_CURRENT_STEP, _LA // 2), jnp.int32),
            pltpu.VMEM((_CHUNK,), jnp.int32),
            pltpu.VMEM((_CHUNK,), jnp.int32),
            pltpu.VMEM((_CHUNK, _D), jnp.float32),
            pltpu.VMEM((_CHUNK, _D // 2), jnp.int32),
            pltpu.VMEM((_CHUNK, _D // 2), jnp.int32),
            pltpu.SemaphoreType.DMA,
            pltpu.SemaphoreType.DMA,
        ],
    )(prompts, tokens, table)

    seq_len = lp + la
    seq_lens = jnp.full((b,), seq_len, dtype=jnp.int32)
    padding_mask = jnp.arange(seq_len)[None, :] >= seq_lens[:, None]
    return embeds, padding_mask, current_step - 1


# speedup vs baseline: 2.1543x; 1.0477x over previous
import math
import numpy as np
import jax
import jax.numpy as jnp
from jax import lax
from jax.experimental import pallas as pl
from jax.experimental.pallas import tpu as pltpu
from jax.experimental.pallas import tpu_sc as plsc

_B = 16
_TOTAL_STEPS = 8
_CURRENT_STEP = 4
_LP = 512
_LA = 1024
_VOCAB = 1024
_D = 1024
_LANES = 16
_CHUNK = 32


def _embed_kernel(prompts, tokens, table, out,
                  ipx, itx, ib0, ib1, acc_v, db0, db1, sem0, sem1, sem_st):
    nc = 2
    wid = lax.axis_index("s") * nc + lax.axis_index("c")
    b = wid // 2
    half = wid % 2
    scale = math.sqrt(float(_D))
    dbufs = (db0, db1)
    ibufs = (ib0, ib1)
    sems = (sem0, sem1)
    himask = jnp.full((_LANES,), np.int32(-65536), dtype=jnp.int32)
    shift16 = jnp.full((_LANES,), 16, dtype=jnp.int32)
    iota2 = lax.iota(jnp.int32, _LANES) * 2

    def stage(dst, idx_v, i, c0):
        for j in range(0, _CHUNK, _LANES):
            dst[pl.ds(j, _LANES)] = idx_v[i, pl.ds(c0 + j, _LANES)]

    def body_row(buf, mode, r, _):
        for k in range(0, _D // 2, _LANES):
            w = buf[r, pl.ds(k, _LANES)]
            lo = lax.bitcast_convert_type(lax.shift_left(w, shift16), jnp.float32)
            hi = lax.bitcast_convert_type(lax.bitwise_and(w, himask), jnp.float32)
            slo = pl.ds(2 * k, _LANES)
            shi = pl.ds(2 * k + _LANES, _LANES)
            if mode == 0:
                acc_v[r, slo] = lo
                acc_v[r, shi] = hi
            elif mode == 1:
                plsc.addupdate(acc_v.at[r, slo], lo)
                plsc.addupdate(acc_v.at[r, shi], hi)
            else:
                acc_v[r, slo] = (acc_v[r, slo] + lo) * scale
                acc_v[r, shi] = (acc_v[r, shi] + hi) * scale
        return 0

    def do_section(idx_hbm, idx_v, num_steps, t0, nrows, out_t0):

        pltpu.sync_copy(idx_hbm.at[b, :, pl.ds(t0, nrows)], idx_v)
        for i in range(1, num_steps):
            off = jnp.full((_LANES,), i * _VOCAB, dtype=jnp.int32)
            for j in range(0, nrows, _LANES):
                sl = pl.ds(j, _LANES)
                idx_v[i, sl] = idx_v[i, sl] + off

        def chunk_body(ci, _):
            c0 = pl.multiple_of(ci * _CHUNK, _CHUNK)
            stage(ibufs[0], idx_v, 0, c0)
            caps = {0: pltpu.async_copy(table.at[ibufs[0]], dbufs[0], sems[0])}
            stage(ibufs[1], idx_v, 1, c0)
            caps[1] = pltpu.async_copy(table.at[ibufs[1]], dbufs[1], sems[1])
            @pl.when(ci > 0)
            def _():
                pltpu.make_async_copy(
                    out.at[b, pl.ds(out_t0 + t0, _CHUNK)], acc_v, sem_st
                ).wait()
            for i in range(num_steps):
                s = i % 2
                caps[i].wait()
                mode = 0 if i == 0 else (2 if i == num_steps - 1 else 1)
                def _pass(r, _buf=dbufs[s], _m=mode):
                    body_row(_buf, _m, r, 0)
                plsc.parallel_loop(0, _CHUNK, step=1)(_pass)
                if i + 2 < num_steps:
                    stage(ibufs[s], idx_v, i + 2, c0)
                    caps[i + 2] = pltpu.async_copy(
                        table.at[ibufs[s]], dbufs[s], sems[s])
            pltpu.async_copy(
                acc_v, out.at[b, pl.ds(out_t0 + t0 + c0, _CHUNK)], sem_st)
            return 0

        lax.fori_loop(0, nrows // _CHUNK, chunk_body, 0)
        pltpu.make_async_copy(
            out.at[b, pl.ds(out_t0 + t0, _CHUNK)], acc_v, sem_st
        ).wait()

    do_section(prompts, ipx, _TOTAL_STEPS, half * (_LP // 2), _LP // 2, 0)
    do_section(tokens, itx, _CURRENT_STEP, half * (_LA // 2), _LA // 2, _LP)


def kernel(acoustic_prompts, acoustic_tokens, a_embeds):
    b, total_steps, lp = acoustic_prompts.shape
    current_step = acoustic_tokens.shape[1]
    la = acoustic_tokens.shape[2]
    d = a_embeds.shape[-1]

    prompts = acoustic_prompts.astype(jnp.int32)
    tokens = acoustic_tokens.astype(jnp.int32)
    v = total_steps * _VOCAB
    tbits = lax.bitcast_convert_type(a_embeds.reshape(v, d), jnp.int32)
    rv = lax.shift_right_logical(
        tbits + 0x7FFF + (lax.shift_right_logical(tbits, 16) & 1), 16)
    rv = rv.reshape(v, d // 32, 2, _LANES)
    table = rv[:, :, 0, :] | lax.shift_left(rv[:, :, 1, :], 16)
    table = table.reshape(v, d // 2)

    mesh = plsc.VectorSubcoreMesh(
        core_axis_name="c", subcore_axis_name="s", num_cores=2, num_subcores=16
    )
    embeds = pl.kernel(
        _embed_kernel,
        out_type=jax.ShapeDtypeStruct((b, lp + la, d), jnp.float32),
        mesh=mesh,
        scratch_types=[
            pltpu.VMEM((_TOTAL_STEPS, _LP // 2), jnp.int32),
            pltpu.VMEM((_CURRENT_STEP, _LA // 2), jnp.int32),
            pltpu.VMEM((_CHUNK,), jnp.int32),
            pltpu.VMEM((_CHUNK,), jnp.int32),
            pltpu.VMEM((_CHUNK, _D), jnp.float32),
            pltpu.VMEM((_CHUNK, _D // 2), jnp.int32),
            pltpu.VMEM((_CHUNK, _D // 2), jnp.int32),
            pltpu.SemaphoreType.DMA,
            pltpu.SemaphoreType.DMA,
            pltpu.SemaphoreType.DMA,
        ],
    )(prompts, tokens, table)

    seq_len = lp + la
    seq_lens = jnp.full((b,), seq_len, dtype=jnp.int32)
    padding_mask = jnp.arange(seq_len)[None, :] >= seq_lens[:, None]
    return embeds, padding_mask, current_step - 1
